# Initial kernel scaffold; baseline (speedup 1.0000x reference)
import jax
import jax.numpy as jnp
from jax.experimental import pallas as pl


def kernel(x, mask_prev, W_enc, b_enc, W_dec, b_dec):
    raise NotImplementedError("write your pallas kernel here")


# speedup vs baseline: 14.3053x; 14.3053x over previous
import functools

import jax
import jax.numpy as jnp
from jax.experimental import pallas as pl
from jax.experimental.pallas import tpu as pltpu

_B, _T = 2, 2048
_IDIM, _ODIM, _HDIM, _CDIM = 1024, 1024, 2048, 128
_N = _B * _T
_BT = 256
_TOP = 0x7F800001


def _body(x_ref, mp_ref, we_ref, be_ref, wd_ref, bd_ref, out_ref, mask_ref):
    x = x_ref[...]
    h = jnp.dot(x, we_ref[...], preferred_element_type=jnp.float32) + be_ref[...]
    h = jnp.where(mp_ref[...] > 0, 0.0, h)
    e = h * h
    ebits = jax.lax.bitcast_convert_type(e, jnp.int32)

    def it(_, c):
        lo1, hi1, lo2, hi2 = c
        mid1 = lo1 + ((hi1 - lo1) >> 1)
        mid2 = lo2 + ((hi2 - lo2) >> 1)
        cnt1 = jnp.sum((ebits >= mid1).astype(jnp.int32), axis=1, keepdims=True)
        cnt2 = jnp.sum((ebits >= mid2).astype(jnp.int32), axis=1, keepdims=True)
        ge1 = cnt1 >= _CDIM
        ge2 = cnt2 >= 2 * _CDIM
        lo1 = jnp.where(ge1, mid1, lo1)
        hi1 = jnp.where(ge1, hi1, mid1)
        lo2 = jnp.where(ge2, mid2, lo2)
        hi2 = jnp.where(ge2, hi2, mid2)
        return lo1, hi1, lo2, hi2

    z = jnp.zeros((_BT, 1), jnp.int32)
    top = jnp.full((_BT, 1), _TOP, jnp.int32)
    lo1, _, lo2, _ = jax.lax.fori_loop(0, 31, it, (z, top, z, top))

    mask_cur = (ebits >= lo1).astype(jnp.float32)
    hm = jnp.where(ebits >= lo2, h, 0.0)
    out_ref[...] = jnp.dot(hm, wd_ref[...], preferred_element_type=jnp.float32) + bd_ref[...]
    mask_ref[...] = mp_ref[...] + mask_cur


@functools.partial(jax.jit, static_argnames=())
def kernel(x, mask_prev, W_enc, b_enc, W_dec, b_dec):
    x2 = x.reshape(_N, _IDIM)
    mp2 = mask_prev.reshape(_N, _HDIM)
    be2 = b_enc.reshape(1, _HDIM)
    bd2 = b_dec.reshape(1, _ODIM)
    grid = (_N // _BT,)
    out, mask = pl.pallas_call(
        _body,
        grid=grid,
        in_specs=[
            pl.BlockSpec((_BT, _IDIM), lambda i: (i, 0)),
            pl.BlockSpec((_BT, _HDIM), lambda i: (i, 0)),
            pl.BlockSpec((_IDIM, _HDIM), lambda i: (0, 0)),
            pl.BlockSpec((1, _HDIM), lambda i: (0, 0)),
            pl.BlockSpec((_HDIM, _ODIM), lambda i: (0, 0)),
            pl.BlockSpec((1, _ODIM), lambda i: (0, 0)),
        ],
        out_specs=[
            pl.BlockSpec((_BT, _ODIM), lambda i: (i, 0)),
            pl.BlockSpec((_BT, _HDIM), lambda i: (i, 0)),
        ],
        out_shape=[
            jax.ShapeDtypeStruct((_N, _ODIM), jnp.float32),
            jax.ShapeDtypeStruct((_N, _HDIM), jnp.float32),
        ],
        compiler_params=pltpu.CompilerParams(
            dimension_semantics=("arbitrary",),
        ),
    )(x2, mp2, W_enc, be2, W_dec, bd2)
    return out.reshape(_B, _T, _ODIM), mask.reshape(_B, _T, _HDIM)
